# Initial kernel scaffold; baseline (speedup 1.0000x reference)
import jax
import jax.numpy as jnp
from jax.experimental import pallas as pl


def kernel(x, edge_index, edge_attr, batch_idx, eps1, eW1, eb1, W1, b1, gn_w, gn_b, gn_ms, eps2, eW2, eb2, W2, b2):
    raise NotImplementedError("write your pallas kernel here")


# speedup vs baseline: 1.1265x; 1.1265x over previous
import jax
import jax.numpy as jnp
from jax.experimental import pallas as pl


def _sigmoid_pallas(x):
    def body(x_ref, o_ref):
        o_ref[...] = jax.nn.sigmoid(x_ref[...])
    return pl.pallas_call(body, out_shape=jax.ShapeDtypeStruct(x.shape, x.dtype))(x)


def kernel(x, edge_index, edge_attr, batch_idx, eps1, eW1, eb1, W1, b1, gn_w, gn_b, gn_ms, eps2, eW2, eb2, W2, b2):
    src, dst = edge_index[0], edge_index[1]
    N = x.shape[0]
    G = 64
    z = jax.nn.relu(x[src] + edge_attr @ eW1 + eb1) @ W1
    agg5 = jax.ops.segment_sum(z, dst, num_segments=N)
    h = (1.0 + eps1) * (x @ W1) + agg5 + b1
    ones = jnp.ones((N,), x.dtype)
    counts = jnp.maximum(jax.ops.segment_sum(ones, batch_idx, num_segments=G), 1.0)[:, None]
    mean = jax.ops.segment_sum(h, batch_idx, num_segments=G) / counts
    sub = h - gn_ms * mean[batch_idx]
    var = jax.ops.segment_sum(sub * sub, batch_idx, num_segments=G) / counts
    h = gn_w * sub / jnp.sqrt(var[batch_idx] + 1e-5) + gn_b
    h = jax.nn.relu(h)
    z2 = jax.nn.relu(h[src] + edge_attr @ eW2 + eb2) @ W2
    agg1 = jax.ops.segment_sum(z2, dst, num_segments=N)
    out = (1.0 + eps2) * (h @ W2) + agg1 + b2
    return _sigmoid_pallas(out)


# speedup vs baseline: 2.7505x; 2.4417x over previous
import functools

import jax
import jax.numpy as jnp
from jax import lax
from jax.experimental import pallas as pl
from jax.experimental.pallas import tpu as pltpu
from jax.experimental.pallas import tpu_sc as plsc

N = 10000
E = 320000
D = 128
G = 64
N2 = 10240
NC = 2
NS = 16
CHUNK = 128
NCHUNK = E // CHUNK
CPC = NCHUNK // NC
CPS_BASE = CPC // NS
CPS_REM = CPC - CPS_BASE * NS


def _f32(*shape):
    return jax.ShapeDtypeStruct(shape, jnp.float32)


def _tc1_body(ea_ref, ew1_ref, eb1_ref, ew2_ref, eb2_ref, p1_ref, p2_ref):
    ea = ea_ref[...]
    p1_ref[...] = jnp.dot(ea, ew1_ref[...], preferred_element_type=jnp.float32) + eb1_ref[...]
    p2_ref[...] = jnp.dot(ea, ew2_ref[...], preferred_element_type=jnp.float32) + eb2_ref[...]


def _tc1(edge_attr, eW1, eb1, eW2p, eb2p):
    BE = 640
    grid = (E // BE,)
    return pl.pallas_call(
        _tc1_body,
        grid=grid,
        in_specs=[
            pl.BlockSpec((BE, 16), lambda i: (i, 0)),
            pl.BlockSpec((16, 128), lambda i: (0, 0)),
            pl.BlockSpec((1, 128), lambda i: (0, 0)),
            pl.BlockSpec((16, 16), lambda i: (0, 0)),
            pl.BlockSpec((1, 16), lambda i: (0, 0)),
        ],
        out_specs=[
            pl.BlockSpec((BE, 128), lambda i: (i, 0)),
            pl.BlockSpec((BE, 16), lambda i: (i, 0)),
        ],
        out_shape=[_f32(E, 128), _f32(E, 16)],
    )(edge_attr, eW1, eb1, eW2p, eb2p)


def _sc_seg_body(width, tab_hbm, p_hbm, src_hbm, dst_hbm, out_hbm,
                 trows, prow, sidx, didx, acc, sem):
    c = lax.axis_index("c")
    s = lax.axis_index("s")
    w16 = width // 16

    def _zrow(r, _):
        for cc in range(128 // 16):
            trows[r, pl.ds(cc * 16, 16)] = jnp.zeros((16,), jnp.float32)
        return _
    lax.fori_loop(0, CHUNK, _zrow, 0)
    row0 = s * (N2 // NS)
    for j in range(N2 // NS // CHUNK):
        pltpu.sync_copy(trows, acc.at[pl.ds(row0 + j * CHUNK, CHUNK), :])
    plsc.subcore_barrier()

    base = c * CPC + s * CPS_BASE + jnp.minimum(s, CPS_REM)
    nchunks = CPS_BASE + jnp.where(s < CPS_REM, 1, 0)

    def _chunk(i, _):
        ebase = (base + i) * CHUNK
        pltpu.sync_copy(src_hbm.at[pl.ds(ebase, CHUNK)], sidx)
        pltpu.sync_copy(dst_hbm.at[pl.ds(ebase, CHUNK)], didx)
        gat = pltpu.async_copy(tab_hbm.at[sidx], trows, sem)
        pltpu.sync_copy(p_hbm.at[pl.ds(ebase, CHUNK), :], prow)
        gat.wait()

        def _row(r, _):
            for cc in range(w16):
                sl = pl.ds(cc * 16, 16)
                trows[r, sl] = jnp.maximum(trows[r, sl] + prow[r, sl], 0.0)
            return _
        lax.fori_loop(0, CHUNK, _row, 0)
        pltpu.sync_copy(trows, acc.at[didx], add=True)
        return _

    lax.fori_loop(0, nchunks, _chunk, 0)
    plsc.subcore_barrier()
    for j in range(N2 // NS // CHUNK):
        r0 = row0 + j * CHUNK
        pltpu.sync_copy(acc.at[pl.ds(r0, CHUNK), :],
                        out_hbm.at[pl.ds(c * N2 + r0, CHUNK), :])


def _sc_seg(table, p, src, dst, width):
    mesh = plsc.VectorSubcoreMesh(core_axis_name="c", subcore_axis_name="s")
    f = pl.kernel(
        functools.partial(_sc_seg_body, width),
        out_type=_f32(NC * N2, 128),
        mesh=mesh,
        scratch_types=[
            pltpu.VMEM((CHUNK, 128), jnp.float32),
            pltpu.VMEM((CHUNK, width), jnp.float32),
            pltpu.VMEM((CHUNK,), jnp.int32),
            pltpu.VMEM((CHUNK,), jnp.int32),
            pltpu.VMEM_SHARED((N2, 128), jnp.float32),
            pltpu.SemaphoreType.DMA,
        ],
    )
    return f(table, p, src, dst)


def _tc2_body(x_ref, a0_ref, a1_ref, bi_ref, w1_ref, b1_ref, gw_ref, gb_ref,
              gm_ref, w2_ref, e1_ref, e2_ref, b2_ref, h8_ref, hw_ref):
    x = x_ref[...]
    agg = a0_ref[...] + a1_ref[...]
    eps1 = e1_ref[0, 0]
    xe = (1.0 + eps1) * x + agg
    h1 = jnp.dot(xe, w1_ref[...], preferred_element_type=jnp.float32) + b1_ref[...]

    bi = bi_ref[...]
    gid = lax.broadcasted_iota(jnp.int32, (N, G), 1)
    oh = (bi == gid).astype(jnp.float32)
    cnt = jnp.maximum(jnp.sum(oh, axis=0, keepdims=True), 1.0)
    ssum = jnp.dot(oh.T, h1, preferred_element_type=jnp.float32)
    mean = ssum / cnt.T
    sub = h1 - gm_ref[...] * jnp.dot(oh, mean, preferred_element_type=jnp.float32)
    svar = jnp.dot(oh.T, sub * sub, preferred_element_type=jnp.float32) / cnt.T
    var = jnp.dot(oh, svar, preferred_element_type=jnp.float32)
    h = gw_ref[...] * sub * lax.rsqrt(var + 1e-5) + gb_ref[...]
    h = jnp.maximum(h, 0.0)
    h8_ref[...] = h
    eps2 = e2_ref[0, 0]
    hw_ref[...] = (1.0 + eps2) * jnp.dot(h, w2_ref[...], preferred_element_type=jnp.float32) + b2_ref[...]


def _tc2(x, a0, a1, bi, W1p, b1p, gwp, gbp, gmp, W2p, eps1, eps2, b2p):
    return pl.pallas_call(
        _tc2_body,
        out_shape=[_f32(N, 128), _f32(N, 16)],
    )(x, a0, a1, bi, W1p, b1p, gwp, gbp, gmp, W2p, eps1, eps2, b2p)


def _tc3_body(hw_ref, g0_ref, g1_ref, w2_ref, o_ref):
    agg = g0_ref[...] + g1_ref[...]
    z = hw_ref[...] + jnp.dot(agg, w2_ref[...], preferred_element_type=jnp.float32)
    o_ref[...] = jax.nn.sigmoid(z)[:, 0:1]


def _tc3(hw2e, g0, g1, W2p):
    return pl.pallas_call(
        _tc3_body,
        out_shape=_f32(N, 1),
    )(hw2e, g0, g1, W2p)


def kernel(x, edge_index, edge_attr, batch_idx, eps1, eW1, eb1, W1, b1,
           gn_w, gn_b, gn_ms, eps2, eW2, eb2, W2, b2):
    f32 = jnp.float32
    src = edge_index[0]
    dst = edge_index[1]

    eW2p = jnp.zeros((16, 16), f32).at[:, :5].set(eW2)
    eb2p = jnp.zeros((1, 16), f32).at[0, :5].set(eb2)
    W1p = jnp.zeros((128, 128), f32).at[:, :5].set(W1)
    b1p = jnp.zeros((1, 128), f32).at[0, :5].set(b1)
    W2p = jnp.zeros((128, 16), f32).at[:5, 0].set(W2[:, 0])
    W2q = jnp.zeros((16, 16), f32).at[:5, 0].set(W2[:, 0])
    b2p = jnp.zeros((1, 16), f32).at[0, 0].set(b2[0])
    gwp = jnp.zeros((1, 128), f32).at[0, :5].set(gn_w)
    gbp = jnp.zeros((1, 128), f32).at[0, :5].set(gn_b)
    gmp = jnp.zeros((1, 128), f32).at[0, :5].set(gn_ms)

    p1, p2 = _tc1(edge_attr, eW1, eb1.reshape(1, 128), eW2p, eb2p)

    part = _sc_seg(x, p1, src, dst, 128)
    a0 = part[:N]
    a1 = part[N2:N2 + N]

    h128, hw2e = _tc2(x, a0, a1, batch_idx.reshape(N, 1), W1p, b1p, gwp, gbp, gmp,
                      W2p, eps1.reshape(1, 1), eps2.reshape(1, 1), b2p)

    part2 = _sc_seg(h128, p2, src, dst, 16)
    g0 = part2[:N, :16]
    g1 = part2[N2:N2 + N, :16]

    return _tc3(hw2e, g0, g1, W2q)


# speedup vs baseline: 3.8903x; 1.4144x over previous
import functools

import jax
import jax.numpy as jnp
from jax import lax
from jax.experimental import pallas as pl
from jax.experimental.pallas import tpu as pltpu
from jax.experimental.pallas import tpu_sc as plsc

N = 10000
E = 320000
D = 128
G = 64
N2 = 10240
NC = 2
NS = 16
CHUNK = 80
CPS = E // (NC * NS) // CHUNK


def _f32(*shape):
    return jax.ShapeDtypeStruct(shape, jnp.float32)


def _tc1_body(ea_ref, ew1_ref, eb1_ref, ew2_ref, eb2_ref, p1_ref, p2_ref):
    ea = ea_ref[...]
    p1_ref[...] = jnp.dot(ea, ew1_ref[...], preferred_element_type=jnp.float32) + eb1_ref[...]
    p2_ref[...] = jnp.dot(ea, ew2_ref[...], preferred_element_type=jnp.float32) + eb2_ref[...]


def _tc1(edge_attr, eW1, eb1, eW2p, eb2p):
    BE = 640
    grid = (E // BE,)
    return pl.pallas_call(
        _tc1_body,
        grid=grid,
        in_specs=[
            pl.BlockSpec((BE, 16), lambda i: (i, 0)),
            pl.BlockSpec((16, 128), lambda i: (0, 0)),
            pl.BlockSpec((1, 128), lambda i: (0, 0)),
            pl.BlockSpec((16, 16), lambda i: (0, 0)),
            pl.BlockSpec((1, 16), lambda i: (0, 0)),
        ],
        out_specs=[
            pl.BlockSpec((BE, 128), lambda i: (i, 0)),
            pl.BlockSpec((BE, 16), lambda i: (i, 0)),
        ],
        out_shape=[_f32(E, 128), _f32(E, 16)],
    )(edge_attr, eW1, eb1, eW2p, eb2p)


def _sc_seg_body(width, tab_hbm, p_hbm, src_hbm, dst_hbm, out_hbm,
                 trows, prow, sidx, didx, acc, gsem, psem, isem):
    c = lax.axis_index("c")
    s = lax.axis_index("s")
    w16 = width // 16

    def _zrow(r, _):
        for cc in range(128 // 16):
            trows[0, r, pl.ds(cc * 16, 16)] = jnp.zeros((16,), jnp.float32)
        return _
    lax.fori_loop(0, CHUNK, _zrow, 0)
    row0 = s * (N2 // NS)
    for j in range(N2 // NS // CHUNK):
        pltpu.sync_copy(trows.at[0], acc.at[pl.ds(row0 + j * CHUNK, CHUNK), :])
    plsc.subcore_barrier()

    base = (c * NS + s) * CPS

    def _issue_idx(cid, slot):
        ebase = cid * CHUNK
        pltpu.async_copy(src_hbm.at[pl.ds(ebase, CHUNK)], sidx.at[slot], isem.at[slot])
        pltpu.async_copy(dst_hbm.at[pl.ds(ebase, CHUNK)], didx.at[slot], isem.at[slot])

    def _wait_idx(slot):
        pltpu.make_async_copy(src_hbm.at[pl.ds(0, CHUNK)], sidx.at[slot], isem.at[slot]).wait()
        pltpu.make_async_copy(dst_hbm.at[pl.ds(0, CHUNK)], didx.at[slot], isem.at[slot]).wait()

    def _pslice(cid):
        return p_hbm.at[pl.ds(cid * CHUNK, CHUNK), :]

    def _issue_gp(cid, dslot, islot):
        pltpu.async_copy(tab_hbm.at[sidx.at[islot]], trows.at[dslot], gsem.at[dslot])
        pltpu.async_copy(_pslice(cid), prow.at[dslot], psem.at[dslot])

    def _wait_gp(dslot):
        pltpu.make_async_copy(tab_hbm.at[sidx.at[0]], trows.at[dslot], gsem.at[dslot]).wait()
        pltpu.make_async_copy(_pslice(0), prow.at[dslot], psem.at[dslot]).wait()

    def _compute_scatter(dslot, islot):
        def _row(r, _):
            for cc in range(w16):
                sl = pl.ds(cc * 16, 16)
                trows[dslot, r, sl] = jnp.maximum(
                    trows[dslot, r, sl] + prow[dslot, r, sl], 0.0)
            return _
        lax.fori_loop(0, CHUNK, _row, 0)
        pltpu.sync_copy(trows.at[dslot], acc.at[didx.at[islot]], add=True)

    _issue_idx(base + 0, 0)
    _issue_idx(base + 1, 1)
    _wait_idx(0)
    _issue_gp(base + 0, 0, 0)

    def _group(g, carry):
        for k in range(4):
            i = g * 4 + k

            @pl.when(i + 1 < CPS)
            def _():
                _wait_idx((k + 1) % 4)
                _issue_gp(base + i + 1, (k + 1) % 2, (k + 1) % 4)

            @pl.when(i + 2 < CPS)
            def _():
                _issue_idx(base + i + 2, (k + 2) % 4)

            _wait_gp(k % 2)
            _compute_scatter(k % 2, k)
        return carry

    lax.fori_loop(0, (CPS - 1) // 4, _group, 0)
    _wait_gp((CPS - 1) % 2)
    _compute_scatter((CPS - 1) % 2, (CPS - 1) % 4)

    plsc.subcore_barrier()
    for j in range(N2 // NS // CHUNK):
        r0 = row0 + j * CHUNK
        pltpu.sync_copy(acc.at[pl.ds(r0, CHUNK), :],
                        out_hbm.at[pl.ds(c * N2 + r0, CHUNK), :])


def _sc_seg(table, p, src, dst, width):
    mesh = plsc.VectorSubcoreMesh(core_axis_name="c", subcore_axis_name="s")
    f = pl.kernel(
        functools.partial(_sc_seg_body, width),
        out_type=_f32(NC * N2, 128),
        mesh=mesh,
        scratch_types=[
            pltpu.VMEM((2, CHUNK, 128), jnp.float32),
            pltpu.VMEM((2, CHUNK, width), jnp.float32),
            pltpu.VMEM((4, CHUNK), jnp.int32),
            pltpu.VMEM((4, CHUNK), jnp.int32),
            pltpu.VMEM_SHARED((N2, 128), jnp.float32),
            pltpu.SemaphoreType.DMA((2,)),
            pltpu.SemaphoreType.DMA((2,)),
            pltpu.SemaphoreType.DMA((4,)),
        ],
    )
    return f(table, p, src, dst)


def _tc2_body(x_ref, a0_ref, a1_ref, bi_ref, w1_ref, b1_ref, gw_ref, gb_ref,
              gm_ref, w2_ref, e1_ref, e2_ref, b2_ref, h8_ref, hw_ref):
    x = x_ref[...]
    agg = a0_ref[...] + a1_ref[...]
    eps1 = e1_ref[0, 0]
    xe = (1.0 + eps1) * x + agg
    h1 = jnp.dot(xe, w1_ref[...], preferred_element_type=jnp.float32) + b1_ref[...]

    bi = bi_ref[...]
    gid = lax.broadcasted_iota(jnp.int32, (N, G), 1)
    oh = (bi == gid).astype(jnp.float32)
    cnt = jnp.maximum(jnp.sum(oh, axis=0, keepdims=True), 1.0)
    ssum = jnp.dot(oh.T, h1, preferred_element_type=jnp.float32)
    mean = ssum / cnt.T
    sub = h1 - gm_ref[...] * jnp.dot(oh, mean, preferred_element_type=jnp.float32)
    svar = jnp.dot(oh.T, sub * sub, preferred_element_type=jnp.float32) / cnt.T
    var = jnp.dot(oh, svar, preferred_element_type=jnp.float32)
    h = gw_ref[...] * sub * lax.rsqrt(var + 1e-5) + gb_ref[...]
    h = jnp.maximum(h, 0.0)
    h8_ref[...] = h
    eps2 = e2_ref[0, 0]
    hw_ref[...] = (1.0 + eps2) * jnp.dot(h, w2_ref[...], preferred_element_type=jnp.float32) + b2_ref[...]


def _tc2(x, a0, a1, bi, W1p, b1p, gwp, gbp, gmp, W2p, eps1, eps2, b2p):
    return pl.pallas_call(
        _tc2_body,
        out_shape=[_f32(N, 128), _f32(N, 16)],
    )(x, a0, a1, bi, W1p, b1p, gwp, gbp, gmp, W2p, eps1, eps2, b2p)


def _tc3_body(hw_ref, g0_ref, g1_ref, w2_ref, o_ref):
    agg = g0_ref[...] + g1_ref[...]
    z = hw_ref[...] + jnp.dot(agg, w2_ref[...], preferred_element_type=jnp.float32)
    o_ref[...] = jax.nn.sigmoid(z)[:, 0:1]


def _tc3(hw2e, g0, g1, W2p):
    return pl.pallas_call(
        _tc3_body,
        out_shape=_f32(N, 1),
    )(hw2e, g0, g1, W2p)


def kernel(x, edge_index, edge_attr, batch_idx, eps1, eW1, eb1, W1, b1,
           gn_w, gn_b, gn_ms, eps2, eW2, eb2, W2, b2):
    f32 = jnp.float32
    src = edge_index[0]
    dst = edge_index[1]

    eW2p = jnp.zeros((16, 16), f32).at[:, :5].set(eW2)
    eb2p = jnp.zeros((1, 16), f32).at[0, :5].set(eb2)
    W1p = jnp.zeros((128, 128), f32).at[:, :5].set(W1)
    b1p = jnp.zeros((1, 128), f32).at[0, :5].set(b1)
    W2p = jnp.zeros((128, 16), f32).at[:5, 0].set(W2[:, 0])
    W2q = jnp.zeros((16, 16), f32).at[:5, 0].set(W2[:, 0])
    b2p = jnp.zeros((1, 16), f32).at[0, 0].set(b2[0])
    gwp = jnp.zeros((1, 128), f32).at[0, :5].set(gn_w)
    gbp = jnp.zeros((1, 128), f32).at[0, :5].set(gn_b)
    gmp = jnp.zeros((1, 128), f32).at[0, :5].set(gn_ms)

    p1, p2 = _tc1(edge_attr, eW1, eb1.reshape(1, 128), eW2p, eb2p)

    part = _sc_seg(x, p1, src, dst, 128)
    a0 = part[:N]
    a1 = part[N2:N2 + N]

    h128, hw2e = _tc2(x, a0, a1, batch_idx.reshape(N, 1), W1p, b1p, gwp, gbp, gmp,
                      W2p, eps1.reshape(1, 1), eps2.reshape(1, 1), b2p)

    part2 = _sc_seg(h128, p2, src, dst, 16)
    g0 = part2[:N, :16]
    g1 = part2[N2:N2 + N, :16]

    return _tc3(hw2e, g0, g1, W2q)
